# baseline (speedup 1.0000x reference)
import jax
import jax.numpy as jnp
from jax import lax
from jax.experimental import pallas as pl
from jax.experimental.pallas import tpu as pltpu
from jax.experimental.pallas import tpu_sc as plsc

BATCH = 16384
LIST = 200
EXAM_N = 201
REL_N = 201 * 201

NC = 2
NS = 16
NW = NC * NS
L = 16

COLS_PER_W = BATCH // NW
RG = 8
NCHUNK = LIST // RG
SLICES = RG * COLS_PER_W // L


def _sigmoid(x):
    return 1.0 / (1.0 + jnp.exp(-x))


def _sigmoid_table_inplace(buf, n, unroll):
    nfull = n // L
    tail_raw = buf[pl.ds(n - L, L)]

    @plsc.parallel_loop(0, nfull, 1, unroll=unroll)
    def _(j):
        x = buf[pl.ds(j * L, L)]
        buf[pl.ds(j * L, L)] = _sigmoid(x)

    buf[pl.ds(n - L, L)] = _sigmoid(tail_raw)


@pl.kernel(
    mesh=plsc.VectorSubcoreMesh(core_axis_name="c", subcore_axis_name="s"),
    out_type=jax.ShapeDtypeStruct((LIST, BATCH), jnp.float32),
    compiler_params=pltpu.CompilerParams(
        needs_layout_passes=False,
        skip_device_barrier=True,
        disable_bounds_checks=True,
        disable_semaphore_checks=True,
    ),
    scratch_types=[
        pltpu.VMEM((EXAM_N,), jnp.float32),
        pltpu.VMEM((REL_N,), jnp.float32),
        pltpu.VMEM((RG, COLS_PER_W), jnp.int32),
        pltpu.VMEM((RG, COLS_PER_W), jnp.int32),
        pltpu.VMEM((RG, COLS_PER_W), jnp.int32),
        pltpu.VMEM((RG, COLS_PER_W), jnp.int32),
        pltpu.VMEM((RG, COLS_PER_W), jnp.int32),
        pltpu.VMEM((RG, COLS_PER_W), jnp.int32),
        pltpu.VMEM((RG, COLS_PER_W), jnp.int32),
        pltpu.VMEM((RG, COLS_PER_W), jnp.int32),
        pltpu.VMEM((RG, COLS_PER_W), jnp.float32),
        pltpu.VMEM((RG, COLS_PER_W), jnp.float32),
        pltpu.VMEM((RG, COLS_PER_W), jnp.float32),
        pltpu.VMEM((RG, COLS_PER_W), jnp.float32),
        pltpu.SemaphoreType.DMA,
        pltpu.SemaphoreType.DMA,
        pltpu.SemaphoreType.DMA,
        pltpu.SemaphoreType.DMA,
        pltpu.SemaphoreType.DMA,
        pltpu.SemaphoreType.DMA,
        pltpu.SemaphoreType.DMA,
        pltpu.SemaphoreType.DMA,
    ],
)
def _all_pairs_pbm(k_hbm, kp_hbm, exam_hbm, rel_hbm, out_hbm,
                   exam_t, rel_t, k0, k1, k2, k3, kp0, kp1, kp2, kp3, o0, o1, o2, o3,
                   isem0, isem1, isem2, isem3, osem0, osem1, osem2, osem3):
    wid = lax.axis_index("s") * NC + lax.axis_index("c")
    col_base = wid * COLS_PER_W
    k_v = (k0, k1, k2, k3)
    kp_v = (kp0, kp1, kp2, kp3)
    o_v = (o0, o1, o2, o3)
    isem = (isem0, isem1, isem2, isem3)
    osem = (osem0, osem1, osem2, osem3)
    NSLOT = 4

    def start_in(c):
        s = c % NSLOT
        r0 = c * RG
        hk = pltpu.async_copy(
            k_hbm.at[pl.ds(r0, RG), pl.ds(col_base, COLS_PER_W)], k_v[s], isem[s])
        hkp = pltpu.async_copy(
            kp_hbm.at[pl.ds(r0, RG), pl.ds(col_base, COLS_PER_W)], kp_v[s], isem[s])
        return hk, hkp

    with jax.named_scope("table_prep"):
        ht1 = pltpu.async_copy(exam_hbm, exam_t, isem[3])
        ht2 = pltpu.async_copy(rel_hbm, rel_t, isem[3])
        in_flight = {0: start_in(0), 1: start_in(1), 2: start_in(2)}
        ht1.wait()
        ht2.wait()
        _sigmoid_table_inplace(exam_t, EXAM_N, unroll=4)
        _sigmoid_table_inplace(rel_t, REL_N, unroll=8)

    out_flight = {}
    for c in range(NCHUNK):
        s = c % NSLOT
        hk, hkp = in_flight.pop(c)
        hk.wait()
        hkp.wait()
        if c + 3 < NCHUNK:
            in_flight[c + 3] = start_in(c + 3)
        if c >= NSLOT:
            out_flight.pop(c - NSLOT).wait()

        kb, kpb, ob = k_v[s], kp_v[s], o_v[s]

        def row_body(r, carry):
            @plsc.parallel_loop(0, COLS_PER_W // L, 1, unroll=16)
            def _(j):
                col = j * L
                kv = kb[r, pl.ds(col, L)]
                kpv = kpb[r, pl.ds(col, L)]
                e = plsc.load_gather(exam_t, [kv])
                g = plsc.load_gather(rel_t, [kv * kpv])
                ob[r, pl.ds(col, L)] = e * g
            return carry

        lax.fori_loop(0, RG, row_body, 0)

        r0 = c * RG
        out_flight[c] = pltpu.async_copy(
            ob, out_hbm.at[pl.ds(r0, RG), pl.ds(col_base, COLS_PER_W)], osem[s])
    for h in out_flight.values():
        h.wait()


def kernel(k, k_prime, exam_table, rel_table):
    out_t = _all_pairs_pbm(k.astype(jnp.int32).T, k_prime.astype(jnp.int32).T,
                           exam_table.reshape(EXAM_N),
                           rel_table.reshape(REL_N))
    return out_t.T


# speedup vs baseline: 1.0618x; 1.0618x over previous
import jax
import jax.numpy as jnp
from jax import lax
from jax.experimental import pallas as pl
from jax.experimental.pallas import tpu as pltpu
from jax.experimental.pallas import tpu_sc as plsc

BATCH = 16384
LIST = 200
EXAM_N = 201
REL_N = 201 * 201

NC = 2
NS = 16
NW = NC * NS
L = 16

COLS_PER_W = BATCH // NW
RG = 8
NCHUNK = LIST // RG
SLICES = RG * COLS_PER_W // L


def _sigmoid(x):
    return 1.0 / (1.0 + jnp.exp(-x))


def _sigmoid_range(buf, start, length, unroll):
    nfull = length // L
    ragged = length - nfull * L
    if ragged:
        tail_raw = buf[pl.ds(start + length - L, L)]

    @plsc.parallel_loop(0, nfull, 1, unroll=unroll)
    def _(j):
        x = buf[pl.ds(start + j * L, L)]
        buf[pl.ds(start + j * L, L)] = _sigmoid(x)

    if ragged:
        buf[pl.ds(start + length - L, L)] = _sigmoid(tail_raw)


REL_SEGS = ((0, 10112), (10112, 10112), (20224, 10112), (30336, REL_N - 30336))


@pl.kernel(
    mesh=plsc.VectorSubcoreMesh(core_axis_name="c", subcore_axis_name="s"),
    out_type=jax.ShapeDtypeStruct((LIST, BATCH), jnp.float32),
    compiler_params=pltpu.CompilerParams(
        needs_layout_passes=False,
        skip_device_barrier=True,
        disable_bounds_checks=True,
        disable_semaphore_checks=True,
    ),
    scratch_types=[
        pltpu.VMEM((EXAM_N,), jnp.float32),
        pltpu.VMEM((REL_N,), jnp.float32),
        pltpu.VMEM((RG, COLS_PER_W), jnp.int32),
        pltpu.VMEM((RG, COLS_PER_W), jnp.int32),
        pltpu.VMEM((RG, COLS_PER_W), jnp.int32),
        pltpu.VMEM((RG, COLS_PER_W), jnp.int32),
        pltpu.VMEM((RG, COLS_PER_W), jnp.int32),
        pltpu.VMEM((RG, COLS_PER_W), jnp.int32),
        pltpu.VMEM((RG, COLS_PER_W), jnp.int32),
        pltpu.VMEM((RG, COLS_PER_W), jnp.int32),
        pltpu.VMEM((RG, COLS_PER_W), jnp.float32),
        pltpu.VMEM((RG, COLS_PER_W), jnp.float32),
        pltpu.VMEM((RG, COLS_PER_W), jnp.float32),
        pltpu.VMEM((RG, COLS_PER_W), jnp.float32),
        pltpu.SemaphoreType.DMA,
        pltpu.SemaphoreType.DMA,
        pltpu.SemaphoreType.DMA,
        pltpu.SemaphoreType.DMA,
        pltpu.SemaphoreType.DMA,
        pltpu.SemaphoreType.DMA,
        pltpu.SemaphoreType.DMA,
        pltpu.SemaphoreType.DMA,
    ],
)
def _all_pairs_pbm(k_hbm, kp_hbm, exam_hbm, rel_hbm, out_hbm,
                   exam_t, rel_t, k0, k1, k2, k3, kp0, kp1, kp2, kp3, o0, o1, o2, o3,
                   isem0, isem1, isem2, isem3, osem0, osem1, osem2, osem3):
    wid = lax.axis_index("s") * NC + lax.axis_index("c")
    col_base = wid * COLS_PER_W
    k_v = (k0, k1, k2, k3)
    kp_v = (kp0, kp1, kp2, kp3)
    o_v = (o0, o1, o2, o3)
    isem = (isem0, isem1, isem2, isem3)
    osem = (osem0, osem1, osem2, osem3)
    NSLOT = 4

    def start_in(c):
        s = c % NSLOT
        r0 = c * RG
        hk = pltpu.async_copy(
            k_hbm.at[pl.ds(r0, RG), pl.ds(col_base, COLS_PER_W)], k_v[s], isem[s])
        hkp = pltpu.async_copy(
            kp_hbm.at[pl.ds(r0, RG), pl.ds(col_base, COLS_PER_W)], kp_v[s], isem[s])
        return hk, hkp

    with jax.named_scope("table_prep"):
        ht_exam = pltpu.async_copy(exam_hbm, exam_t, isem[3])
        ht_rel = [
            pltpu.async_copy(rel_hbm.at[pl.ds(s0, ln)],
                             rel_t.at[pl.ds(s0, ln)], isem[3])
            for s0, ln in REL_SEGS
        ]
        in_flight = {0: start_in(0), 1: start_in(1), 2: start_in(2)}
        ht_exam.wait()
        _sigmoid_range(exam_t, 0, EXAM_N, unroll=4)
        for (s0, ln), h in zip(REL_SEGS, ht_rel):
            h.wait()
            _sigmoid_range(rel_t, s0, ln, unroll=8)

    out_flight = {}
    for c in range(NCHUNK):
        s = c % NSLOT
        hk, hkp = in_flight.pop(c)
        hk.wait()
        hkp.wait()
        if c + 3 < NCHUNK:
            in_flight[c + 3] = start_in(c + 3)
        if c >= NSLOT:
            out_flight.pop(c - NSLOT).wait()

        kb, kpb, ob = k_v[s], kp_v[s], o_v[s]

        def row_body(r, carry):
            @plsc.parallel_loop(0, COLS_PER_W // L, 1, unroll=8)
            def _(j):
                col = j * L
                kv = kb[r, pl.ds(col, L)]
                kpv = kpb[r, pl.ds(col, L)]
                e = plsc.load_gather(exam_t, [kv])
                g = plsc.load_gather(rel_t, [kv * kpv])
                ob[r, pl.ds(col, L)] = e * g
            return carry

        lax.fori_loop(0, RG, row_body, 0)

        r0 = c * RG
        out_flight[c] = pltpu.async_copy(
            ob, out_hbm.at[pl.ds(r0, RG), pl.ds(col_base, COLS_PER_W)], osem[s])
    for h in out_flight.values():
        h.wait()


def kernel(k, k_prime, exam_table, rel_table):
    out_t = _all_pairs_pbm(k.astype(jnp.int32).T, k_prime.astype(jnp.int32).T,
                           exam_table.reshape(EXAM_N),
                           rel_table.reshape(REL_N))
    return out_t.T
